# Initial kernel scaffold; baseline (speedup 1.0000x reference)
import jax
import jax.numpy as jnp
from jax.experimental import pallas as pl


def kernel(xyz, params):
    raise NotImplementedError("write your pallas kernel here")


# speedup vs baseline: 2.4399x; 2.4399x over previous
import functools

import numpy as np
import jax
import jax.numpy as jnp
from jax.experimental import pallas as pl

_BN_SCALE = np.float32(1.0 / np.sqrt(1.0 + 1e-5))
_BIG = np.float32(1e30)


def _fold_conv_bn(conv, bn):
    s = _BN_SCALE * bn["g"]
    wt = (conv["W"] * s[:, None]).T
    b = conv["b"] * s + bn["b"]
    return wt.astype(jnp.float32), b[None, :].astype(jnp.float32)


def _density_body(N, CH, bw):
    alpha = np.float32(1.0 / (2.0 * bw * bw))
    const = np.float32(N * 2.5 * bw)
    nchunk = N // CH

    def body(rows_ref, cols_ref, out_ref):
        xyz = rows_ref[0]
        nall = jnp.sum(xyz * xyz, axis=0, keepdims=True)
        acc = jnp.zeros((1, N), jnp.float32)
        for i in range(nchunk):
            lhs = cols_ref[0, i * CH:(i + 1) * CH, :]
            g = jax.lax.dot_general(lhs, xyz, (((1,), (0,)), ((), ())),
                                    preferred_element_type=jnp.float32)
            nl = jnp.sum(lhs * lhs, axis=1, keepdims=True)
            d = nl + nall - 2.0 * g
            acc = acc + jnp.sum(jnp.exp(d * (-alpha)), axis=0, keepdims=True)
        out_ref[0] = const / acc

    return body


def _density(rows, cols, bw, ch):
    B, _, N = rows.shape
    body = _density_body(N, ch, bw)
    return pl.pallas_call(
        body,
        grid=(B,),
        in_specs=[
            pl.BlockSpec((1, 3, N), lambda b: (b, 0, 0)),
            pl.BlockSpec((1, N, 3), lambda b: (b, 0, 0)),
        ],
        out_specs=pl.BlockSpec((1, 1, N), lambda b: (b, 0, 0)),
        out_shape=jax.ShapeDtypeStruct((B, 1, N), jnp.float32),
    )(rows, cols)


def _fps_body(N, S):
    def body(rows_ref, out_ref):
        xyz = rows_ref[0]
        lane_n = jax.lax.broadcasted_iota(jnp.int32, (1, N), 1)
        lane_s = jax.lax.broadcasted_iota(jnp.int32, (1, S), 1)

        def step(s, state):
            dist, far, acc = state
            mask = lane_n == far
            cent = jnp.sum(jnp.where(mask, xyz, 0.0), axis=1, keepdims=True)
            acc = jnp.where(lane_s == s, cent, acc)
            d = jnp.sum((xyz - cent) ** 2, axis=0, keepdims=True)
            dist = jnp.minimum(dist, d)
            m = jnp.max(dist, axis=1, keepdims=True)
            far = jnp.min(jnp.where(dist == m, lane_n, N), axis=1,
                          keepdims=True)
            return dist, far, acc

        init = (jnp.full((1, N), 1e10, jnp.float32),
                jnp.zeros((1, 1), jnp.int32),
                jnp.zeros((3, S), jnp.float32))
        _, _, acc = jax.lax.fori_loop(0, S, step, init)
        out_ref[0] = acc

    return body


def _fps(rows, S):
    B, _, N = rows.shape
    return pl.pallas_call(
        _fps_body(N, S),
        grid=(B,),
        in_specs=[pl.BlockSpec((1, 3, N), lambda b: (b, 0, 0))],
        out_specs=pl.BlockSpec((1, 3, S), lambda b: (b, 0, 0)),
        out_shape=jax.ShapeDtypeStruct((B, 3, S), jnp.float32),
    )(rows)


def _group_body(N, S, K, CF, Cmid):
    def body(cols_ref, rows_ref, feats_ref,
             w1t_r, b1_r, wd1_r, bd1_r, wd2t_r, bd2_r, wd3r_r, bd3_r,
             ww1t_r, bw1_r, ww2t_r, bw2_r, ww3t_r, bw3_r, wlt_r, bl_r,
             out_ref):
        sq = cols_ref[0]
        xyz = rows_ref[0]
        feats = feats_ref[0]

        g = jax.lax.dot_general(sq, xyz, (((1,), (0,)), ((), ())),
                                preferred_element_type=jnp.float32)
        d = (jnp.sum(sq * sq, axis=1, keepdims=True)
             + jnp.sum(xyz * xyz, axis=0, keepdims=True) - 2.0 * g)

        iota = jax.lax.broadcasted_iota(jnp.int32, (S, N), 1)
        dist = d
        gk, dk = [], []
        for _ in range(K):
            m = jnp.min(dist, axis=1, keepdims=True)
            idxk = jnp.min(jnp.where(dist == m, iota, N), axis=1,
                           keepdims=True)
            sel = iota == idxk
            oh = sel.astype(jnp.float32)
            gr = jax.lax.dot_general(oh, feats, (((1,), (0,)), ((), ())),
                                     preferred_element_type=jnp.float32)
            gk.append(gr)
            dk.append(gr[:, CF - 1:CF])
            dist = jnp.where(sel, _BIG, dist)

        dmax = functools.reduce(jnp.maximum, dk)
        ds_col = jnp.concatenate([x / dmax for x in dk], axis=0)
        xg = jnp.concatenate(gk, axis=0)
        sqrep = jnp.concatenate([sq] * K, axis=0)
        xn = xg[:, 0:3] - sqrep
        pgrp = xg[:, 3:CF - 1]

        relu = jax.nn.relu
        dot = lambda a, b: jnp.dot(a, b, preferred_element_type=jnp.float32)

        h0 = jnp.concatenate([xn, pgrp], axis=1)
        h1 = relu(dot(h0, w1t_r[...]) + b1_r[...])

        s1 = relu(ds_col * wd1_r[...] + bd1_r[...])
        s2 = relu(dot(s1, wd2t_r[...]) + bd2_r[...])
        s3 = relu(jnp.sum(s2 * wd3r_r[...], axis=1, keepdims=True)
                  + bd3_r[...])
        h = h1 * s3

        t1 = relu(dot(xn, ww1t_r[...]) + bw1_r[...])
        t2 = relu(dot(t1, ww2t_r[...]) + bw2_r[...])
        t3 = relu(dot(t2, ww3t_r[...]) + bw3_r[...])

        ca = jax.lax.broadcasted_iota(jnp.int32, (Cmid, 16 * Cmid), 1)
        ra = jax.lax.broadcasted_iota(jnp.int32, (Cmid, 16 * Cmid), 0)
        rmat = ((ca // 16) == ra).astype(jnp.float32)
        cb = jax.lax.broadcasted_iota(jnp.int32, (16, 16 * Cmid), 1)
        rb = jax.lax.broadcasted_iota(jnp.int32, (16, 16 * Cmid), 0)
        tmat = ((cb % 16) == rb).astype(jnp.float32)

        mm = jnp.zeros((S, 16 * Cmid), jnp.float32)
        for k in range(K):
            hk = h[k * S:(k + 1) * S, :]
            wk = t3[k * S:(k + 1) * S, :]
            mm = mm + dot(hk, rmat) * dot(wk, tmat)

        out_ref[0] = relu(dot(mm, wlt_r[...]) + bl_r[...])

    return body


def _group(cols, rows, feats, wts, S, K, Cmid):
    B, _, N = rows.shape
    CF = feats.shape[-1]
    body = _group_body(N, S, K, CF, Cmid)
    wspecs = [pl.BlockSpec(w.shape, lambda b, nd=w.ndim: (0,) * nd)
              for w in wts]
    return pl.pallas_call(
        body,
        grid=(B,),
        in_specs=[
            pl.BlockSpec((1, S, 3), lambda b: (b, 0, 0)),
            pl.BlockSpec((1, 3, N), lambda b: (b, 0, 0)),
            pl.BlockSpec((1, N, CF), lambda b: (b, 0, 0)),
        ] + wspecs,
        out_specs=pl.BlockSpec((1, S, Cmid), lambda b: (b, 0, 0)),
        out_shape=jax.ShapeDtypeStruct((B, S, Cmid), jnp.float32),
    )(cols, rows, feats, *wts)


def _layer3_body(N, Cpts, Cout, bw):
    alpha = np.float32(1.0 / (2.0 * bw * bw))
    const = np.float32(N * 2.5 * bw)

    def body(cols_ref, rows_ref, pts_ref,
             w1t_r, b1_r, wd1_r, bd1_r, wd2t_r, bd2_r, wd3r_r, bd3_r,
             ww1t_r, bw1_r, ww2t_r, bw2_r, ww3t_r, bw3_r, wl3t_r, bl3_r,
             out_ref):
        cols = cols_ref[0]
        rows = rows_ref[0]
        pts = pts_ref[0]

        g = jax.lax.dot_general(cols, rows, (((1,), (0,)), ((), ())),
                                preferred_element_type=jnp.float32)
        d = (jnp.sum(cols * cols, axis=1, keepdims=True)
             + jnp.sum(rows * rows, axis=0, keepdims=True) - 2.0 * g)
        sexp = jnp.sum(jnp.exp(d * (-alpha)), axis=1, keepdims=True)
        invd = const / sexp
        ds = invd / jnp.max(invd, axis=0, keepdims=True)

        relu = jax.nn.relu
        dot = lambda a, b: jnp.dot(a, b, preferred_element_type=jnp.float32)

        s1 = relu(ds * wd1_r[...] + bd1_r[...])
        s2 = relu(dot(s1, wd2t_r[...]) + bd2_r[...])
        s3 = relu(jnp.sum(s2 * wd3r_r[...], axis=1, keepdims=True)
                  + bd3_r[...])

        h0 = jnp.concatenate([cols, pts], axis=1)
        h = relu(dot(h0, w1t_r[...]) + b1_r[...]) * s3

        t1 = relu(dot(cols, ww1t_r[...]) + bw1_r[...])
        t2 = relu(dot(t1, ww2t_r[...]) + bw2_r[...])
        t3 = relu(dot(t2, ww3t_r[...]) + bw3_r[...])

        mmt = jax.lax.dot_general(t3, h, (((0,), (0,)), ((), ())),
                                  preferred_element_type=jnp.float32)
        mmf = mmt.reshape(1, 16 * Cout)
        out_ref[0] = relu(dot(mmf, wl3t_r[...]) + bl3_r[...])

    return body


def _layer3(cols, rows, pts, wts, Cout, bw):
    B, _, N = rows.shape
    Cpts = pts.shape[-1]
    body = _layer3_body(N, Cpts, Cout, bw)
    wspecs = [pl.BlockSpec(w.shape, lambda b, nd=w.ndim: (0,) * nd)
              for w in wts]
    return pl.pallas_call(
        body,
        grid=(B,),
        in_specs=[
            pl.BlockSpec((1, N, 3), lambda b: (b, 0, 0)),
            pl.BlockSpec((1, 3, N), lambda b: (b, 0, 0)),
            pl.BlockSpec((1, N, Cpts), lambda b: (b, 0, 0)),
        ] + wspecs,
        out_specs=pl.BlockSpec((1, 1, Cout), lambda b: (b, 0, 0)),
        out_shape=jax.ShapeDtypeStruct((B, 1, Cout), jnp.float32),
    )(cols, rows, pts, *wts)


def _sa_weights(p):
    w1t, b1 = _fold_conv_bn(p["mlp_convs"][0], p["mlp_bns"][0])
    dn, wn = p["densitynet"], p["weightnet"]
    wd1t, bd1 = _fold_conv_bn(dn["convs"][0], dn["bns"][0])
    wd2t, bd2 = _fold_conv_bn(dn["convs"][1], dn["bns"][1])
    wd3t, bd3 = _fold_conv_bn(dn["convs"][2], dn["bns"][2])
    ww1t, bw1 = _fold_conv_bn(wn["convs"][0], wn["bns"][0])
    ww2t, bw2 = _fold_conv_bn(wn["convs"][1], wn["bns"][1])
    ww3t, bw3 = _fold_conv_bn(wn["convs"][2], wn["bns"][2])
    sl = _BN_SCALE * p["bn_linear"]["g"]
    wl = p["linear"]["W"] * sl[:, None]
    bl = (p["linear"]["b"] * sl + p["bn_linear"]["b"])[None, :]
    return (w1t, b1, wd1t, bd1, wd2t, bd2, wd3t.T, bd3,
            ww1t, bw1, ww2t, bw2, ww3t, bw3, wl, bl)


def kernel(xyz, params):
    B, _, N = xyz.shape
    f32 = jnp.float32
    rows1 = xyz.astype(f32)
    cols1 = jnp.swapaxes(rows1, 1, 2)

    w1 = _sa_weights(params["sa1"])
    wts1 = w1[:14] + (jnp.swapaxes(w1[14], 0, 1), w1[15])
    invd1 = _density(rows1, cols1, 0.1, 256)
    feats1 = jnp.concatenate(
        [cols1, cols1, jnp.reshape(invd1, (B, N, 1))], axis=-1)
    nxr1 = _fps(rows1, 128)
    nxc1 = jnp.swapaxes(nxr1, 1, 2)
    pts1 = _group(nxc1, rows1, feats1, wts1, 128, 8, 32)

    w2 = _sa_weights(params["sa2"])
    wts2 = w2[:14] + (jnp.swapaxes(w2[14], 0, 1), w2[15])
    invd2 = _density(nxr1, nxc1, 0.2, 128)
    feats2 = jnp.concatenate(
        [nxc1, pts1, jnp.reshape(invd2, (B, 128, 1))], axis=-1)
    nxr2 = _fps(nxr1, 64)
    nxc2 = jnp.swapaxes(nxr2, 1, 2)
    pts2 = _group(nxc2, nxr1, feats2, wts2, 64, 16, 64)

    w3 = _sa_weights(params["sa3"])
    wl3 = w3[14]
    wl3t = jnp.reshape(
        jnp.transpose(jnp.reshape(wl3, (128, 128, 16)), (2, 1, 0)),
        (2048, 128))
    wts3 = w3[:14] + (wl3t, w3[15])
    out3 = _layer3(nxc2, nxr2, pts2, wts3, 128, 0.4)
    return jnp.reshape(out3, (B, 128))


# speedup vs baseline: 6.3987x; 2.6226x over previous
import functools

import numpy as np
import jax
import jax.numpy as jnp
from jax.experimental import pallas as pl

_BN_SCALE = np.float32(1.0 / np.sqrt(1.0 + 1e-5))
_BIG = np.float32(1e30)


def _fold_conv_bn(conv, bn):
    s = _BN_SCALE * bn["g"]
    wt = (conv["W"] * s[:, None]).T
    b = conv["b"] * s + bn["b"]
    return wt.astype(jnp.float32), b[None, :].astype(jnp.float32)


def _density_body(N, CH, bw):
    alpha = np.float32(1.0 / (2.0 * bw * bw))
    const = np.float32(N * 2.5 * bw)
    nchunk = N // CH

    def body(rows_ref, cols_ref, out_ref):
        xyz = rows_ref[0]
        nall = jnp.sum(xyz * xyz, axis=0, keepdims=True)
        acc = jnp.zeros((1, N), jnp.float32)
        for i in range(nchunk):
            lhs = cols_ref[0, i * CH:(i + 1) * CH, :]
            g = jax.lax.dot_general(lhs, xyz, (((1,), (0,)), ((), ())),
                                    preferred_element_type=jnp.float32)
            nl = jnp.sum(lhs * lhs, axis=1, keepdims=True)
            d = nl + nall - 2.0 * g
            acc = acc + jnp.sum(jnp.exp(d * (-alpha)), axis=0, keepdims=True)
        out_ref[0] = const / acc

    return body


def _density(rows, cols, bw, ch):
    B, _, N = rows.shape
    body = _density_body(N, ch, bw)
    return pl.pallas_call(
        body,
        grid=(B,),
        in_specs=[
            pl.BlockSpec((1, 3, N), lambda b: (b, 0, 0)),
            pl.BlockSpec((1, N, 3), lambda b: (b, 0, 0)),
        ],
        out_specs=pl.BlockSpec((1, 1, N), lambda b: (b, 0, 0)),
        out_shape=jax.ShapeDtypeStruct((B, 1, N), jnp.float32),
    )(rows, cols)


def _fps_body(B, N, S):
    def body(planes_ref, out_ref):
        x = planes_ref[0]
        y = planes_ref[1]
        z = planes_ref[2]
        lane_n = jax.lax.broadcasted_iota(jnp.int32, (1, N), 1)
        lane_s = jax.lax.broadcasted_iota(jnp.int32, (1, S), 1)

        def step(s, state):
            dist, far, ax, ay, az = state
            mask = lane_n == far
            cx = jnp.sum(jnp.where(mask, x, 0.0), axis=1, keepdims=True)
            cy = jnp.sum(jnp.where(mask, y, 0.0), axis=1, keepdims=True)
            cz = jnp.sum(jnp.where(mask, z, 0.0), axis=1, keepdims=True)
            sel = lane_s == s
            ax = jnp.where(sel, cx, ax)
            ay = jnp.where(sel, cy, ay)
            az = jnp.where(sel, cz, az)
            d = (x - cx) ** 2 + (y - cy) ** 2 + (z - cz) ** 2
            dist = jnp.minimum(dist, d)
            m = jnp.max(dist, axis=1, keepdims=True)
            far = jnp.min(jnp.where(dist == m, lane_n, N), axis=1,
                          keepdims=True)
            return dist, far, ax, ay, az

        init = (jnp.full((B, N), 1e10, jnp.float32),
                jnp.zeros((B, 1), jnp.int32),
                jnp.zeros((B, S), jnp.float32),
                jnp.zeros((B, S), jnp.float32),
                jnp.zeros((B, S), jnp.float32))
        _, _, ax, ay, az = jax.lax.fori_loop(0, S, step, init)
        out_ref[0] = ax
        out_ref[1] = ay
        out_ref[2] = az

    return body


def _fps(rows, S):
    B, _, N = rows.shape
    planes = jnp.swapaxes(rows, 0, 1)
    out = pl.pallas_call(
        _fps_body(B, N, S),
        in_specs=[pl.BlockSpec((3, B, N), lambda: (0, 0, 0))],
        out_specs=pl.BlockSpec((3, B, S), lambda: (0, 0, 0)),
        out_shape=jax.ShapeDtypeStruct((3, B, S), jnp.float32),
    )(planes)
    return jnp.swapaxes(out, 0, 1)


def _group_body(N, S, K, CF, Cmid):
    def body(cols_ref, rows_ref, feats_ref,
             w1t_r, b1_r, wd1_r, bd1_r, wd2t_r, bd2_r, wd3r_r, bd3_r,
             ww1t_r, bw1_r, ww2t_r, bw2_r, ww3t_r, bw3_r, wlt_r, bl_r,
             out_ref):
        sq = cols_ref[0]
        xyz = rows_ref[0]
        feats = feats_ref[0]

        g = jax.lax.dot_general(sq, xyz, (((1,), (0,)), ((), ())),
                                preferred_element_type=jnp.float32)
        d = (jnp.sum(sq * sq, axis=1, keepdims=True)
             + jnp.sum(xyz * xyz, axis=0, keepdims=True) - 2.0 * g)

        iota = jax.lax.broadcasted_iota(jnp.int32, (S, N), 1)
        dist = d
        gk, dk = [], []
        for _ in range(K):
            m = jnp.min(dist, axis=1, keepdims=True)
            idxk = jnp.min(jnp.where(dist == m, iota, N), axis=1,
                           keepdims=True)
            sel = iota == idxk
            oh = sel.astype(jnp.float32)
            gr = jax.lax.dot_general(oh, feats, (((1,), (0,)), ((), ())),
                                     preferred_element_type=jnp.float32)
            gk.append(gr)
            dk.append(gr[:, CF - 1:CF])
            dist = jnp.where(sel, _BIG, dist)

        dmax = functools.reduce(jnp.maximum, dk)
        ds_col = jnp.concatenate([x / dmax for x in dk], axis=0)
        xg = jnp.concatenate(gk, axis=0)
        sqrep = jnp.concatenate([sq] * K, axis=0)
        xn = xg[:, 0:3] - sqrep
        pgrp = xg[:, 3:CF - 1]

        relu = jax.nn.relu
        dot = lambda a, b: jnp.dot(a, b, preferred_element_type=jnp.float32)

        h0 = jnp.concatenate([xn, pgrp], axis=1)
        h1 = relu(dot(h0, w1t_r[...]) + b1_r[...])

        s1 = relu(ds_col * wd1_r[...] + bd1_r[...])
        s2 = relu(dot(s1, wd2t_r[...]) + bd2_r[...])
        s3 = relu(jnp.sum(s2 * wd3r_r[...], axis=1, keepdims=True)
                  + bd3_r[...])
        h = h1 * s3

        t1 = relu(dot(xn, ww1t_r[...]) + bw1_r[...])
        t2 = relu(dot(t1, ww2t_r[...]) + bw2_r[...])
        t3 = relu(dot(t2, ww3t_r[...]) + bw3_r[...])

        ca = jax.lax.broadcasted_iota(jnp.int32, (Cmid, 16 * Cmid), 1)
        ra = jax.lax.broadcasted_iota(jnp.int32, (Cmid, 16 * Cmid), 0)
        rmat = ((ca // 16) == ra).astype(jnp.float32)
        cb = jax.lax.broadcasted_iota(jnp.int32, (16, 16 * Cmid), 1)
        rb = jax.lax.broadcasted_iota(jnp.int32, (16, 16 * Cmid), 0)
        tmat = ((cb % 16) == rb).astype(jnp.float32)

        mm = jnp.zeros((S, 16 * Cmid), jnp.float32)
        for k in range(K):
            hk = h[k * S:(k + 1) * S, :]
            wk = t3[k * S:(k + 1) * S, :]
            mm = mm + dot(hk, rmat) * dot(wk, tmat)

        out_ref[0] = relu(dot(mm, wlt_r[...]) + bl_r[...])

    return body


def _group(cols, rows, feats, wts, S, K, Cmid):
    B, _, N = rows.shape
    CF = feats.shape[-1]
    body = _group_body(N, S, K, CF, Cmid)
    wspecs = [pl.BlockSpec(w.shape, lambda b, nd=w.ndim: (0,) * nd)
              for w in wts]
    return pl.pallas_call(
        body,
        grid=(B,),
        in_specs=[
            pl.BlockSpec((1, S, 3), lambda b: (b, 0, 0)),
            pl.BlockSpec((1, 3, N), lambda b: (b, 0, 0)),
            pl.BlockSpec((1, N, CF), lambda b: (b, 0, 0)),
        ] + wspecs,
        out_specs=pl.BlockSpec((1, S, Cmid), lambda b: (b, 0, 0)),
        out_shape=jax.ShapeDtypeStruct((B, S, Cmid), jnp.float32),
    )(cols, rows, feats, *wts)


def _layer3_body(N, Cpts, Cout, bw):
    alpha = np.float32(1.0 / (2.0 * bw * bw))
    const = np.float32(N * 2.5 * bw)

    def body(cols_ref, rows_ref, pts_ref,
             w1t_r, b1_r, wd1_r, bd1_r, wd2t_r, bd2_r, wd3r_r, bd3_r,
             ww1t_r, bw1_r, ww2t_r, bw2_r, ww3t_r, bw3_r, wl3t_r, bl3_r,
             out_ref):
        cols = cols_ref[0]
        rows = rows_ref[0]
        pts = pts_ref[0]

        g = jax.lax.dot_general(cols, rows, (((1,), (0,)), ((), ())),
                                preferred_element_type=jnp.float32)
        d = (jnp.sum(cols * cols, axis=1, keepdims=True)
             + jnp.sum(rows * rows, axis=0, keepdims=True) - 2.0 * g)
        sexp = jnp.sum(jnp.exp(d * (-alpha)), axis=1, keepdims=True)
        invd = const / sexp
        ds = invd / jnp.max(invd, axis=0, keepdims=True)

        relu = jax.nn.relu
        dot = lambda a, b: jnp.dot(a, b, preferred_element_type=jnp.float32)

        s1 = relu(ds * wd1_r[...] + bd1_r[...])
        s2 = relu(dot(s1, wd2t_r[...]) + bd2_r[...])
        s3 = relu(jnp.sum(s2 * wd3r_r[...], axis=1, keepdims=True)
                  + bd3_r[...])

        h0 = jnp.concatenate([cols, pts], axis=1)
        h = relu(dot(h0, w1t_r[...]) + b1_r[...]) * s3

        t1 = relu(dot(cols, ww1t_r[...]) + bw1_r[...])
        t2 = relu(dot(t1, ww2t_r[...]) + bw2_r[...])
        t3 = relu(dot(t2, ww3t_r[...]) + bw3_r[...])

        mmt = jax.lax.dot_general(t3, h, (((0,), (0,)), ((), ())),
                                  preferred_element_type=jnp.float32)
        mmf = mmt.reshape(1, 16 * Cout)
        out_ref[0] = relu(dot(mmf, wl3t_r[...]) + bl3_r[...])

    return body


def _layer3(cols, rows, pts, wts, Cout, bw):
    B, _, N = rows.shape
    Cpts = pts.shape[-1]
    body = _layer3_body(N, Cpts, Cout, bw)
    wspecs = [pl.BlockSpec(w.shape, lambda b, nd=w.ndim: (0,) * nd)
              for w in wts]
    return pl.pallas_call(
        body,
        grid=(B,),
        in_specs=[
            pl.BlockSpec((1, N, 3), lambda b: (b, 0, 0)),
            pl.BlockSpec((1, 3, N), lambda b: (b, 0, 0)),
            pl.BlockSpec((1, N, Cpts), lambda b: (b, 0, 0)),
        ] + wspecs,
        out_specs=pl.BlockSpec((1, 1, Cout), lambda b: (b, 0, 0)),
        out_shape=jax.ShapeDtypeStruct((B, 1, Cout), jnp.float32),
    )(cols, rows, pts, *wts)


def _sa_weights(p):
    w1t, b1 = _fold_conv_bn(p["mlp_convs"][0], p["mlp_bns"][0])
    dn, wn = p["densitynet"], p["weightnet"]
    wd1t, bd1 = _fold_conv_bn(dn["convs"][0], dn["bns"][0])
    wd2t, bd2 = _fold_conv_bn(dn["convs"][1], dn["bns"][1])
    wd3t, bd3 = _fold_conv_bn(dn["convs"][2], dn["bns"][2])
    ww1t, bw1 = _fold_conv_bn(wn["convs"][0], wn["bns"][0])
    ww2t, bw2 = _fold_conv_bn(wn["convs"][1], wn["bns"][1])
    ww3t, bw3 = _fold_conv_bn(wn["convs"][2], wn["bns"][2])
    sl = _BN_SCALE * p["bn_linear"]["g"]
    wl = p["linear"]["W"] * sl[:, None]
    bl = (p["linear"]["b"] * sl + p["bn_linear"]["b"])[None, :]
    return (w1t, b1, wd1t, bd1, wd2t, bd2, wd3t.T, bd3,
            ww1t, bw1, ww2t, bw2, ww3t, bw3, wl, bl)


def kernel(xyz, params):
    B, _, N = xyz.shape
    f32 = jnp.float32
    rows1 = xyz.astype(f32)
    cols1 = jnp.swapaxes(rows1, 1, 2)

    w1 = _sa_weights(params["sa1"])
    wts1 = w1[:14] + (jnp.swapaxes(w1[14], 0, 1), w1[15])
    invd1 = _density(rows1, cols1, 0.1, 256)
    feats1 = jnp.concatenate(
        [cols1, cols1, jnp.reshape(invd1, (B, N, 1))], axis=-1)
    nxr1 = _fps(rows1, 128)
    nxc1 = jnp.swapaxes(nxr1, 1, 2)
    pts1 = _group(nxc1, rows1, feats1, wts1, 128, 8, 32)

    w2 = _sa_weights(params["sa2"])
    wts2 = w2[:14] + (jnp.swapaxes(w2[14], 0, 1), w2[15])
    invd2 = _density(nxr1, nxc1, 0.2, 128)
    feats2 = jnp.concatenate(
        [nxc1, pts1, jnp.reshape(invd2, (B, 128, 1))], axis=-1)
    nxr2 = _fps(nxr1, 64)
    nxc2 = jnp.swapaxes(nxr2, 1, 2)
    pts2 = _group(nxc2, nxr1, feats2, wts2, 64, 16, 64)

    w3 = _sa_weights(params["sa3"])
    wl3 = w3[14]
    wl3t = jnp.reshape(
        jnp.transpose(jnp.reshape(wl3, (128, 128, 16)), (2, 1, 0)),
        (2048, 128))
    wts3 = w3[:14] + (wl3t, w3[15])
    out3 = _layer3(nxc2, nxr2, pts2, wts3, 128, 0.4)
    return jnp.reshape(out3, (B, 128))


# speedup vs baseline: 6.9888x; 1.0922x over previous
import functools

import numpy as np
import jax
import jax.numpy as jnp
from jax.experimental import pallas as pl

_BN_SCALE = np.float32(1.0 / np.sqrt(1.0 + 1e-5))
_BIG = np.float32(1e30)


def _fold_conv_bn(conv, bn):
    s = _BN_SCALE * bn["g"]
    wt = (conv["W"] * s[:, None]).T
    b = conv["b"] * s + bn["b"]
    return wt.astype(jnp.float32), b[None, :].astype(jnp.float32)


def _inv_density(rows, pf, CH, bw):
    N = rows.shape[1]
    alpha = np.float32(1.0 / (2.0 * bw * bw))
    const = np.float32(N * 2.5 * bw)
    nalls = alpha * jnp.sum(rows * rows, axis=0, keepdims=True)
    parts = []
    for i in range(N // CH):
        chunk = pf[i * CH:(i + 1) * CH, 0:3]
        cs = chunk * alpha
        gs = jax.lax.dot_general(cs, rows, (((1,), (0,)), ((), ())),
                                 preferred_element_type=jnp.float32)
        nls = jnp.sum(cs * chunk, axis=1, keepdims=True)
        arg = 2.0 * gs - nls - nalls
        parts.append(jnp.sum(jnp.exp(arg), axis=1, keepdims=True))
    acc = jnp.concatenate(parts, axis=0) if len(parts) > 1 else parts[0]
    return const / acc


def _fps_body(B, N, S):
    def body(planes_ref, out_ref):
        x = planes_ref[0]
        y = planes_ref[1]
        z = planes_ref[2]
        lane_n = jax.lax.broadcasted_iota(jnp.int32, (1, N), 1)
        lane_s = jax.lax.broadcasted_iota(jnp.int32, (1, S), 1)

        def step(s, state):
            dist, far, ax, ay, az = state
            mask = lane_n == far
            cx = jnp.sum(jnp.where(mask, x, 0.0), axis=1, keepdims=True)
            cy = jnp.sum(jnp.where(mask, y, 0.0), axis=1, keepdims=True)
            cz = jnp.sum(jnp.where(mask, z, 0.0), axis=1, keepdims=True)
            sel = lane_s == s
            ax = jnp.where(sel, cx, ax)
            ay = jnp.where(sel, cy, ay)
            az = jnp.where(sel, cz, az)
            d = (x - cx) ** 2 + (y - cy) ** 2 + (z - cz) ** 2
            dist = jnp.minimum(dist, d)
            m = jnp.max(dist, axis=1, keepdims=True)
            far = jnp.min(jnp.where(dist == m, lane_n, N), axis=1,
                          keepdims=True)
            return dist, far, ax, ay, az

        init = (jnp.full((B, N), 1e10, jnp.float32),
                jnp.zeros((B, 1), jnp.int32),
                jnp.zeros((B, S), jnp.float32),
                jnp.zeros((B, S), jnp.float32),
                jnp.zeros((B, S), jnp.float32))
        _, _, ax, ay, az = jax.lax.fori_loop(0, S, step, init)
        out_ref[0] = ax
        out_ref[1] = ay
        out_ref[2] = az

    return body


def _fps(rows, S):
    B, _, N = rows.shape
    planes = jnp.swapaxes(rows, 0, 1)
    out = pl.pallas_call(
        _fps_body(B, N, S),
        in_specs=[pl.BlockSpec((3, B, N), lambda: (0, 0, 0))],
        out_specs=pl.BlockSpec((3, B, S), lambda: (0, 0, 0)),
        out_shape=jax.ShapeDtypeStruct((3, B, S), jnp.float32),
    )(planes)
    return jnp.swapaxes(out, 0, 1)


def _group_body(N, S, K, CF, Cmid, CH, bw):
    def body(cols_ref, rows_ref, pf_ref,
             w1t_r, b1_r, wd1_r, bd1_r, wd2t_r, bd2_r, wd3r_r, bd3_r,
             ww1t_r, bw1_r, ww2t_r, bw2_r, ww3t_r, bw3_r, wlt_r, bl_r,
             out_ref):
        sq = cols_ref[0]
        xyz = rows_ref[0]
        pf = pf_ref[0]

        invd = _inv_density(xyz, pf, CH, bw)
        feats = jnp.concatenate([invd, pf], axis=1)

        g = jax.lax.dot_general(sq, xyz, (((1,), (0,)), ((), ())),
                                preferred_element_type=jnp.float32)
        d = (jnp.sum(sq * sq, axis=1, keepdims=True)
             + jnp.sum(xyz * xyz, axis=0, keepdims=True) - 2.0 * g)

        iota = jax.lax.broadcasted_iota(jnp.int32, (S, N), 1)
        dist = d
        gk = []
        for _ in range(K):
            m = jnp.min(dist, axis=1, keepdims=True)
            idxk = jnp.min(jnp.where(dist == m, iota, N), axis=1,
                           keepdims=True)
            sel = iota == idxk
            oh = sel.astype(jnp.float32)
            gr = jax.lax.dot_general(oh, feats, (((1,), (0,)), ((), ())),
                                     preferred_element_type=jnp.float32)
            gk.append(gr)
            dist = jnp.where(sel, _BIG, dist)

        dk = [x[:, 0:1] for x in gk]
        dmax = functools.reduce(jnp.maximum, dk)
        ds_col = jnp.concatenate([x / dmax for x in dk], axis=0)
        xg = jnp.concatenate(gk, axis=0)
        sqrep = jnp.concatenate([sq] * K, axis=0)
        xn = xg[:, 1:4] - sqrep
        pgrp = xg[:, 4:CF]

        relu = jax.nn.relu
        dot = lambda a, b: jnp.dot(a, b, preferred_element_type=jnp.float32)

        h0 = jnp.concatenate([xn, pgrp], axis=1)
        h1 = relu(dot(h0, w1t_r[...]) + b1_r[...])

        s1 = relu(ds_col * wd1_r[...] + bd1_r[...])
        s2 = relu(dot(s1, wd2t_r[...]) + bd2_r[...])
        s3 = relu(jnp.sum(s2 * wd3r_r[...], axis=1, keepdims=True)
                  + bd3_r[...])
        h = h1 * s3

        t1 = relu(dot(xn, ww1t_r[...]) + bw1_r[...])
        t2 = relu(dot(t1, ww2t_r[...]) + bw2_r[...])
        t3 = relu(dot(t2, ww3t_r[...]) + bw3_r[...])

        ca = jax.lax.broadcasted_iota(jnp.int32, (Cmid, 16 * Cmid), 1)
        ra = jax.lax.broadcasted_iota(jnp.int32, (Cmid, 16 * Cmid), 0)
        rmat = ((ca // 16) == ra).astype(jnp.float32)
        cb = jax.lax.broadcasted_iota(jnp.int32, (16, 16 * Cmid), 1)
        rb = jax.lax.broadcasted_iota(jnp.int32, (16, 16 * Cmid), 0)
        tmat = ((cb % 16) == rb).astype(jnp.float32)

        mm = jnp.zeros((S, 16 * Cmid), jnp.float32)
        for k in range(K):
            hk = h[k * S:(k + 1) * S, :]
            wk = t3[k * S:(k + 1) * S, :]
            mm = mm + dot(hk, rmat) * dot(wk, tmat)

        out_ref[0] = relu(dot(mm, wlt_r[...]) + bl_r[...])

    return body


def _group(cols, rows, pf, wts, S, K, Cmid, CH, bw):
    B, _, N = rows.shape
    CF = pf.shape[-1] + 1
    body = _group_body(N, S, K, CF, Cmid, CH, bw)
    wspecs = [pl.BlockSpec(w.shape, lambda b, nd=w.ndim: (0,) * nd)
              for w in wts]
    return pl.pallas_call(
        body,
        grid=(B,),
        in_specs=[
            pl.BlockSpec((1, S, 3), lambda b: (b, 0, 0)),
            pl.BlockSpec((1, 3, N), lambda b: (b, 0, 0)),
            pl.BlockSpec((1, N, CF - 1), lambda b: (b, 0, 0)),
        ] + wspecs,
        out_specs=pl.BlockSpec((1, S, Cmid), lambda b: (b, 0, 0)),
        out_shape=jax.ShapeDtypeStruct((B, S, Cmid), jnp.float32),
    )(cols, rows, pf, *wts)


def _layer3_body(N, Cpts, Cout, bw):
    alpha = np.float32(1.0 / (2.0 * bw * bw))
    const = np.float32(N * 2.5 * bw)

    def body(cols_ref, rows_ref, pts_ref,
             w1t_r, b1_r, wd1_r, bd1_r, wd2t_r, bd2_r, wd3r_r, bd3_r,
             ww1t_r, bw1_r, ww2t_r, bw2_r, ww3t_r, bw3_r, wl3t_r, bl3_r,
             out_ref):
        cols = cols_ref[0]
        rows = rows_ref[0]
        pts = pts_ref[0]

        g = jax.lax.dot_general(cols, rows, (((1,), (0,)), ((), ())),
                                preferred_element_type=jnp.float32)
        d = (jnp.sum(cols * cols, axis=1, keepdims=True)
             + jnp.sum(rows * rows, axis=0, keepdims=True) - 2.0 * g)
        sexp = jnp.sum(jnp.exp(d * (-alpha)), axis=1, keepdims=True)
        invd = const / sexp
        ds = invd / jnp.max(invd, axis=0, keepdims=True)

        relu = jax.nn.relu
        dot = lambda a, b: jnp.dot(a, b, preferred_element_type=jnp.float32)

        s1 = relu(ds * wd1_r[...] + bd1_r[...])
        s2 = relu(dot(s1, wd2t_r[...]) + bd2_r[...])
        s3 = relu(jnp.sum(s2 * wd3r_r[...], axis=1, keepdims=True)
                  + bd3_r[...])

        h0 = jnp.concatenate([cols, pts], axis=1)
        h = relu(dot(h0, w1t_r[...]) + b1_r[...]) * s3

        t1 = relu(dot(cols, ww1t_r[...]) + bw1_r[...])
        t2 = relu(dot(t1, ww2t_r[...]) + bw2_r[...])
        t3 = relu(dot(t2, ww3t_r[...]) + bw3_r[...])

        mmt = jax.lax.dot_general(t3, h, (((0,), (0,)), ((), ())),
                                  preferred_element_type=jnp.float32)
        mmf = mmt.reshape(1, 16 * Cout)
        out_ref[0] = relu(dot(mmf, wl3t_r[...]) + bl3_r[...])

    return body


def _layer3(cols, rows, pts, wts, Cout, bw):
    B, _, N = rows.shape
    Cpts = pts.shape[-1]
    body = _layer3_body(N, Cpts, Cout, bw)
    wspecs = [pl.BlockSpec(w.shape, lambda b, nd=w.ndim: (0,) * nd)
              for w in wts]
    return pl.pallas_call(
        body,
        grid=(B,),
        in_specs=[
            pl.BlockSpec((1, N, 3), lambda b: (b, 0, 0)),
            pl.BlockSpec((1, 3, N), lambda b: (b, 0, 0)),
            pl.BlockSpec((1, N, Cpts), lambda b: (b, 0, 0)),
        ] + wspecs,
        out_specs=pl.BlockSpec((1, 1, Cout), lambda b: (b, 0, 0)),
        out_shape=jax.ShapeDtypeStruct((B, 1, Cout), jnp.float32),
    )(cols, rows, pts, *wts)


def _sa_weights(p):
    w1t, b1 = _fold_conv_bn(p["mlp_convs"][0], p["mlp_bns"][0])
    dn, wn = p["densitynet"], p["weightnet"]
    wd1t, bd1 = _fold_conv_bn(dn["convs"][0], dn["bns"][0])
    wd2t, bd2 = _fold_conv_bn(dn["convs"][1], dn["bns"][1])
    wd3t, bd3 = _fold_conv_bn(dn["convs"][2], dn["bns"][2])
    ww1t, bw1 = _fold_conv_bn(wn["convs"][0], wn["bns"][0])
    ww2t, bw2 = _fold_conv_bn(wn["convs"][1], wn["bns"][1])
    ww3t, bw3 = _fold_conv_bn(wn["convs"][2], wn["bns"][2])
    sl = _BN_SCALE * p["bn_linear"]["g"]
    wl = p["linear"]["W"] * sl[:, None]
    bl = (p["linear"]["b"] * sl + p["bn_linear"]["b"])[None, :]
    return (w1t, b1, wd1t, bd1, wd2t, bd2, wd3t.T, bd3,
            ww1t, bw1, ww2t, bw2, ww3t, bw3, wl, bl)


def kernel(xyz, params):
    B, _, N = xyz.shape
    f32 = jnp.float32
    rows1 = xyz.astype(f32)
    cols1 = jnp.swapaxes(rows1, 1, 2)

    w1 = _sa_weights(params["sa1"])
    wts1 = w1[:14] + (jnp.swapaxes(w1[14], 0, 1), w1[15])
    pf1 = jnp.concatenate([cols1, cols1], axis=-1)
    nxr1 = _fps(rows1, 128)
    nxc1 = jnp.swapaxes(nxr1, 1, 2)
    pts1 = _group(nxc1, rows1, pf1, wts1, 128, 8, 32, 256, 0.1)

    w2 = _sa_weights(params["sa2"])
    wts2 = w2[:14] + (jnp.swapaxes(w2[14], 0, 1), w2[15])
    pf2 = jnp.concatenate([nxc1, pts1], axis=-1)
    nxr2 = _fps(nxr1, 64)
    nxc2 = jnp.swapaxes(nxr2, 1, 2)
    pts2 = _group(nxc2, nxr1, pf2, wts2, 64, 16, 64, 128, 0.2)

    w3 = _sa_weights(params["sa3"])
    wl3 = w3[14]
    wl3t = jnp.reshape(
        jnp.transpose(jnp.reshape(wl3, (128, 128, 16)), (2, 1, 0)),
        (2048, 128))
    wts3 = w3[:14] + (wl3t, w3[15])
    out3 = _layer3(nxc2, nxr2, pts2, wts3, 128, 0.4)
    return jnp.reshape(out3, (B, 128))
